# Initial kernel scaffold; baseline (speedup 1.0000x reference)
import jax
import jax.numpy as jnp
from jax.experimental import pallas as pl


def kernel(q, k):
    raise NotImplementedError("write your pallas kernel here")


# speedup vs baseline: 4.5832x; 4.5832x over previous
import math

import jax
import jax.numpy as jnp
from jax.experimental import pallas as pl

_R = 256
_ROWS_PER_STEP = 8


def _select_body(v_ref, o_ref):
    v = v_ref[...]
    b = jax.lax.bitcast_convert_type(v, jnp.int32)
    key = b ^ ((b >> 31) & jnp.int32(0x7FFFFFFF))

    def body(i, t):
        cand = t ^ (jnp.int32(1) << (jnp.int32(31) - i))
        cnt = jnp.sum((key >= cand).astype(jnp.int32), axis=-1, keepdims=True)
        return jnp.where(cnt >= _R, cand, t)

    t0 = jnp.full((_ROWS_PER_STEP, 1), jnp.int32(-2147483648))
    t = jax.lax.fori_loop(0, 32, body, t0)
    mask = key >= t
    lane = jax.lax.broadcasted_iota(jnp.int32, mask.shape, 1)
    o_ref[...] = mask | (lane == 0)


def kernel(q, k):
    d = q.shape[-1]
    a = jnp.matmul(q[..., :1, :], jnp.swapaxes(k[..., 1:, :], -2, -1))
    a = a / math.sqrt(d)
    a = jax.nn.softmax(a, axis=-1)[..., 0, :]
    a2 = a.reshape(-1, a.shape[-1])
    g = jax.random.gumbel(jax.random.key(42), a2.shape, a2.dtype)
    v = jnp.log(a2 + 1e-20) + g
    bsz = v.shape[0]
    v8 = jnp.concatenate(
        [jnp.full((bsz, 1), -jnp.inf, jnp.float32), v], axis=1
    )
    seq = v8.shape[1]
    out = pl.pallas_call(
        _select_body,
        grid=(bsz // _ROWS_PER_STEP,),
        in_specs=[pl.BlockSpec((_ROWS_PER_STEP, seq), lambda i: (i, 0))],
        out_specs=pl.BlockSpec((_ROWS_PER_STEP, seq), lambda i: (i, 0)),
        out_shape=jax.ShapeDtypeStruct((bsz, seq), jnp.bool_),
    )(v8)
    return out
